# baseline (speedup 1.0000x reference)
import math

import jax
import jax.numpy as jnp
from jax import lax
from jax.experimental import pallas as pl
from jax.experimental.pallas import tpu as pltpu

B, T, N, D = 8, 12, 512, 64
NADJ = 16
S_FACTOR = 5
SAMPLES = int(S_FACTOR * math.log(N, 2))
SPAD = 48

_HI = jax.lax.Precision.HIGHEST
_DEF = jax.lax.Precision.DEFAULT


def _dot(a, b, prec=_DEF):
    return lax.dot_general(a, b, (((1,), (0,)), ((), ())),
                           precision=prec, preferred_element_type=jnp.float32)


def _dot_nt(a, b, prec=_DEF):
    return lax.dot_general(a, b, (((1,), (1,)), ((), ())),
                           precision=prec, preferred_element_type=jnp.float32)


def _ln(x, eps=1e-5):
    mu = jnp.mean(x, axis=-1, keepdims=True)
    var = jnp.mean((x - mu) ** 2, axis=-1, keepdims=True)
    return (x - mu) / jnp.sqrt(var + eps)


def _c_kernel(adj_ref, c_ref):
    jI = lax.broadcasted_iota(jnp.int32, (N, N), 1)
    acc = jnp.zeros((N, N), jnp.float32)
    for k in range(NADJ):
        a = adj_ref[:, k][:, None]
        acc = acc + (a == jI).astype(jnp.float32)
    c_ref[...] = acc


def _fused_kernel(x_ref, c_ref, ident_ref, evT_ref, ev_ref,
                  wq_ref, bq_ref, wk_ref, bk_ref, wv_ref, bv_ref,
                  wo_ref, bo_ref, wp_ref, bp_ref,
                  wf1_ref, bf1_ref, wf2_ref, bf2_ref,
                  out_ref):
    ident = ident_ref[...]

    x = x_ref[0]
    identd = (lax.broadcasted_iota(jnp.int32, (D, D), 0)
              == lax.broadcasted_iota(jnp.int32, (D, D), 1)).astype(jnp.float32)
    diagm = identd * ev_ref[...]
    pe = _dot(evT_ref[...], diagm)
    x_ = x + pe

    Q = _dot(x_, wq_ref[...]) + bq_ref[...]
    K = _dot(x_, wk_ref[...]) + bk_ref[...]
    V = _dot(x_, wv_ref[...]) + bv_ref[...]

    S_hi = _dot_nt(Q, K, _HI)
    S_def = _dot_nt(Q, K)
    gat = _dot(S_hi * c_ref[...], V, _HI)

    m_row = _dot_nt(wp_ref[...], gat) + bp_ref[...]
    m_col = _dot_nt(ident, m_row, _HI)

    mj = jnp.broadcast_to(m_col, (N, N))
    mn = jnp.broadcast_to(m_row, (N, N))
    ij = lax.broadcasted_iota(jnp.int32, (N, N), 0)
    inn = lax.broadcasted_iota(jnp.int32, (N, N), 1)
    gt = (mj > mn) | ((mj == mn) & (ij < inn))
    r_row = jnp.sum(gt.astype(jnp.float32), axis=0, keepdims=True)

    iota_s = lax.broadcasted_iota(jnp.int32, (SPAD, N), 0).astype(jnp.float32)
    p = ((iota_s == jnp.broadcast_to(r_row, (SPAD, N)))
         & (iota_s < SAMPLES)).astype(jnp.float32)

    qk = _dot(p, S_def, _HI) * (1.0 / math.sqrt(D))
    qk_max = jnp.max(qk, axis=-1, keepdims=True)
    e = jnp.exp(qk - qk_max)
    attn = e / jnp.sum(e, axis=-1, keepdims=True)

    attn_m = jnp.where(iota_s < SAMPLES, attn, -1.0)
    mx = jnp.max(attn_m, axis=0, keepdims=True)
    cp_row = jnp.min(jnp.where(attn_m == mx, iota_s, float(N)),
                     axis=0, keepdims=True)

    val = _dot(attn, V)

    cp_col = _dot_nt(ident, cp_row, _HI)
    iota_g = lax.broadcasted_iota(jnp.int32, (N, SPAD), 1).astype(jnp.float32)
    g = (jnp.broadcast_to(cp_col, (N, SPAD)) == iota_g).astype(jnp.float32)
    value = _dot(g, val, _HI)

    o1 = _dot(value, wo_ref[...]) + bo_ref[...] + x_
    o1 = _ln(o1)
    h = jnp.maximum(_dot(o1, wf1_ref[...]) + bf1_ref[...], 0.0)
    h = _dot(h, wf2_ref[...]) + bf2_ref[...]
    out_ref[0] = _ln(h + o1)


def kernel(x, adj, eigvec, eigvalue, Wq, bq, Wk, bk, Wv, bv, Wo, bo,
           Wp, bp, Wf1, bf1, Wf2, bf2):
    BT = B * T
    xr = x.reshape(BT, N, D)
    evT = jnp.transpose(eigvec[:, :, 0])
    ev = eigvalue.reshape(1, D)

    full = lambda shape: pl.BlockSpec(shape, lambda i: (0,) * len(shape))
    w = lambda: full((D, D))
    b = lambda: full((1, D))

    C = pl.pallas_call(
        _c_kernel,
        out_shape=jax.ShapeDtypeStruct((N, N), jnp.float32),
    )(adj)
    ident = jnp.eye(N, dtype=jnp.float32)

    out = pl.pallas_call(
        _fused_kernel,
        grid=(BT,),
        in_specs=[
            pl.BlockSpec((1, N, D), lambda i: (i, 0, 0)),
            full((N, N)),
            full((N, N)),
            full((N, D)),
            full((1, D)),
            w(), b(),
            w(), b(),
            w(), b(),
            w(), b(),
            full((1, D)), full((1, 1)),
            w(), b(),
            w(), b(),
        ],
        out_specs=pl.BlockSpec((1, N, D), lambda i: (i, 0, 0)),
        out_shape=jax.ShapeDtypeStruct((BT, N, D), jnp.float32),
        compiler_params=pltpu.CompilerParams(
            dimension_semantics=("parallel",)),
    )(xr, C, ident, evT, ev,
      Wq.T, bq.reshape(1, D), Wk.T, bk.reshape(1, D), Wv.T, bv.reshape(1, D),
      Wo.T, bo.reshape(1, D), Wp, bp.reshape(1, 1),
      Wf1.T, bf1.reshape(1, D), Wf2.T, bf2.reshape(1, D))
    return out.reshape(B, T, N, D)


# speedup vs baseline: 1.0186x; 1.0186x over previous
import math

import jax
import jax.numpy as jnp
from jax import lax
from jax.experimental import pallas as pl
from jax.experimental.pallas import tpu as pltpu

B, T, N, D = 8, 12, 512, 64
NADJ = 16
S_FACTOR = 5
SAMPLES = int(S_FACTOR * math.log(N, 2))
SPAD = 48

_HI = jax.lax.Precision.HIGHEST
_DEF = jax.lax.Precision.DEFAULT


def _dot(a, b, prec=_DEF):
    return lax.dot_general(a, b, (((1,), (0,)), ((), ())),
                           precision=prec, preferred_element_type=jnp.float32)


def _dot_nt(a, b, prec=_DEF):
    return lax.dot_general(a, b, (((1,), (1,)), ((), ())),
                           precision=prec, preferred_element_type=jnp.float32)


def _ln(x, eps=1e-5):
    mu = jnp.mean(x, axis=-1, keepdims=True)
    var = jnp.mean((x - mu) ** 2, axis=-1, keepdims=True)
    return (x - mu) / jnp.sqrt(var + eps)


def _c_kernel(adj_ref, c_ref):
    jI = lax.broadcasted_iota(jnp.int32, (N, N), 1)
    acc = jnp.zeros((N, N), jnp.float32)
    for k in range(NADJ):
        a = adj_ref[:, k][:, None]
        acc = acc + (a == jI).astype(jnp.float32)
    c_ref[...] = acc


UNROLL = 2


def _fused_kernel(x_ref, c_ref, ident_ref, evT_ref, ev_ref,
                  wq_ref, bq_ref, wk_ref, bk_ref, wv_ref, bv_ref,
                  wo_ref, bo_ref, wp_ref, bp_ref,
                  wf1_ref, bf1_ref, wf2_ref, bf2_ref,
                  out_ref):
    ident = ident_ref[...]
    for u in range(UNROLL):
        _one_slice(u, x_ref, c_ref, ident, evT_ref, ev_ref,
                   wq_ref, bq_ref, wk_ref, bk_ref, wv_ref, bv_ref,
                   wo_ref, bo_ref, wp_ref, bp_ref,
                   wf1_ref, bf1_ref, wf2_ref, bf2_ref, out_ref)


def _one_slice(u, x_ref, c_ref, ident, evT_ref, ev_ref,
               wq_ref, bq_ref, wk_ref, bk_ref, wv_ref, bv_ref,
               wo_ref, bo_ref, wp_ref, bp_ref,
               wf1_ref, bf1_ref, wf2_ref, bf2_ref, out_ref):
    x = x_ref[u]
    identd = (lax.broadcasted_iota(jnp.int32, (D, D), 0)
              == lax.broadcasted_iota(jnp.int32, (D, D), 1)).astype(jnp.float32)
    diagm = identd * ev_ref[...]
    pe = _dot(evT_ref[...], diagm)
    x_ = x + pe

    Q = _dot(x_, wq_ref[...]) + bq_ref[...]
    K = _dot(x_, wk_ref[...]) + bk_ref[...]
    V = _dot(x_, wv_ref[...]) + bv_ref[...]

    S_hi = _dot_nt(Q, K, _HI)
    S_def = _dot_nt(Q, K)
    gat = _dot(S_hi * c_ref[...], V, _HI)

    m_row = _dot_nt(wp_ref[...], gat) + bp_ref[...]
    m_col = _dot_nt(ident, m_row, _HI)

    mj = jnp.broadcast_to(m_col, (N, N))
    mn = jnp.broadcast_to(m_row, (N, N))
    ij = lax.broadcasted_iota(jnp.int32, (N, N), 0)
    inn = lax.broadcasted_iota(jnp.int32, (N, N), 1)
    gt = (mj > mn) | ((mj == mn) & (ij < inn))
    r_row = jnp.sum(gt.astype(jnp.float32), axis=0, keepdims=True)

    iota_s = lax.broadcasted_iota(jnp.int32, (SPAD, N), 0).astype(jnp.float32)
    p = ((iota_s == jnp.broadcast_to(r_row, (SPAD, N)))
         & (iota_s < SAMPLES)).astype(jnp.float32)

    qk = _dot(p, S_def, _HI) * (1.0 / math.sqrt(D))
    qk_max = jnp.max(qk, axis=-1, keepdims=True)
    e = jnp.exp(qk - qk_max)
    attn = e / jnp.sum(e, axis=-1, keepdims=True)

    attn_m = jnp.where(iota_s < SAMPLES, attn, -1.0)
    mx = jnp.max(attn_m, axis=0, keepdims=True)
    cp_row = jnp.min(jnp.where(attn_m == mx, iota_s, float(N)),
                     axis=0, keepdims=True)

    val = _dot(attn, V)

    cp_col = _dot_nt(ident, cp_row, _HI)
    iota_g = lax.broadcasted_iota(jnp.int32, (N, SPAD), 1).astype(jnp.float32)
    g = (jnp.broadcast_to(cp_col, (N, SPAD)) == iota_g).astype(jnp.float32)
    value = _dot(g, val, _HI)

    o1 = _dot(value, wo_ref[...]) + bo_ref[...] + x_
    o1 = _ln(o1)
    h = jnp.maximum(_dot(o1, wf1_ref[...]) + bf1_ref[...], 0.0)
    h = _dot(h, wf2_ref[...]) + bf2_ref[...]
    out_ref[u] = _ln(h + o1)


def kernel(x, adj, eigvec, eigvalue, Wq, bq, Wk, bk, Wv, bv, Wo, bo,
           Wp, bp, Wf1, bf1, Wf2, bf2):
    BT = B * T
    xr = x.reshape(BT, N, D)
    evT = jnp.transpose(eigvec[:, :, 0])
    ev = eigvalue.reshape(1, D)

    full = lambda shape: pl.BlockSpec(shape, lambda i: (0,) * len(shape))
    w = lambda: full((D, D))
    b = lambda: full((1, D))

    C = pl.pallas_call(
        _c_kernel,
        out_shape=jax.ShapeDtypeStruct((N, N), jnp.float32),
    )(adj)
    ident = jnp.eye(N, dtype=jnp.float32)

    out = pl.pallas_call(
        _fused_kernel,
        grid=(BT // UNROLL,),
        in_specs=[
            pl.BlockSpec((UNROLL, N, D), lambda i: (i, 0, 0)),
            full((N, N)),
            full((N, N)),
            full((N, D)),
            full((1, D)),
            w(), b(),
            w(), b(),
            w(), b(),
            w(), b(),
            full((1, D)), full((1, 1)),
            w(), b(),
            w(), b(),
        ],
        out_specs=pl.BlockSpec((UNROLL, N, D), lambda i: (i, 0, 0)),
        out_shape=jax.ShapeDtypeStruct((BT, N, D), jnp.float32),
        compiler_params=pltpu.CompilerParams(
            dimension_semantics=("parallel",)),
    )(xr, C, ident, evT, ev,
      Wq.T, bq.reshape(1, D), Wk.T, bk.reshape(1, D), Wv.T, bv.reshape(1, D),
      Wo.T, bo.reshape(1, D), Wp, bp.reshape(1, 1),
      Wf1.T, bf1.reshape(1, D), Wf2.T, bf2.reshape(1, D))
    return out.reshape(B, T, N, D)


# speedup vs baseline: 1.0939x; 1.0739x over previous
import math

import jax
import jax.numpy as jnp
from jax import lax
from jax.experimental import pallas as pl
from jax.experimental.pallas import tpu as pltpu

B, T, N, D = 8, 12, 512, 64
NADJ = 16
S_FACTOR = 5
SAMPLES = int(S_FACTOR * math.log(N, 2))
SPAD = 48

_HI = jax.lax.Precision.HIGHEST
_DEF = jax.lax.Precision.DEFAULT


def _dot(a, b, prec=_DEF):
    return lax.dot_general(a, b, (((1,), (0,)), ((), ())),
                           precision=prec, preferred_element_type=jnp.float32)


def _dot_nt(a, b, prec=_DEF):
    return lax.dot_general(a, b, (((1,), (1,)), ((), ())),
                           precision=prec, preferred_element_type=jnp.float32)


def _ln(x, eps=1e-5):
    mu = jnp.mean(x, axis=-1, keepdims=True)
    var = jnp.mean((x - mu) ** 2, axis=-1, keepdims=True)
    return (x - mu) / jnp.sqrt(var + eps)


def _c_kernel(adj_ref, c_ref):
    jI = lax.broadcasted_iota(jnp.int32, (N, N), 1)
    acc = jnp.zeros((N, N), jnp.float32)
    for k in range(NADJ):
        a = adj_ref[:, k][:, None]
        acc = acc + (a == jI).astype(jnp.float32)
    c_ref[...] = acc


UNROLL = 2


def _fused_kernel(x_ref, c_ref, ident_ref, evT_ref, ev_ref,
                  wq_ref, bq_ref, wk_ref, bk_ref, wv_ref, bv_ref,
                  wo_ref, bo_ref, wp_ref, bp_ref,
                  wf1_ref, bf1_ref, wf2_ref, bf2_ref,
                  out_ref):
    ident = ident_ref[...]
    for u in range(UNROLL):
        _one_slice(u, x_ref, c_ref, ident, evT_ref, ev_ref,
                   wq_ref, bq_ref, wk_ref, bk_ref, wv_ref, bv_ref,
                   wo_ref, bo_ref, wp_ref, bp_ref,
                   wf1_ref, bf1_ref, wf2_ref, bf2_ref, out_ref)


def _one_slice(u, x_ref, c_ref, ident, evT_ref, ev_ref,
               wq_ref, bq_ref, wk_ref, bk_ref, wv_ref, bv_ref,
               wo_ref, bo_ref, wp_ref, bp_ref,
               wf1_ref, bf1_ref, wf2_ref, bf2_ref, out_ref):
    x = x_ref[u]
    identd = (lax.broadcasted_iota(jnp.int32, (D, D), 0)
              == lax.broadcasted_iota(jnp.int32, (D, D), 1)).astype(jnp.float32)
    diagm = identd * ev_ref[...]
    pe = _dot(evT_ref[...], diagm)
    x_ = x + pe

    Q = _dot(x_, wq_ref[...]) + bq_ref[...]
    K = _dot(x_, wk_ref[...]) + bk_ref[...]
    V = _dot(x_, wv_ref[...]) + bv_ref[...]

    S_hi = _dot_nt(Q, K, _HI)
    gat = _dot(S_hi * c_ref[...], V, _HI)

    m_row = _dot_nt(wp_ref[...], gat) + bp_ref[...]
    m_col = _dot_nt(ident, m_row, _HI)

    mj = jnp.broadcast_to(m_col, (N, N))
    mn = jnp.broadcast_to(m_row, (N, N))
    ij = lax.broadcasted_iota(jnp.int32, (N, N), 0)
    inn = lax.broadcasted_iota(jnp.int32, (N, N), 1)
    gt = (mj > mn) | ((mj == mn) & (ij < inn))
    r_row = _dot(jnp.ones((1, N), jnp.float32),
                 gt.astype(jnp.float32))

    iota_s = lax.broadcasted_iota(jnp.int32, (SPAD, N), 0).astype(jnp.float32)
    p = ((iota_s == jnp.broadcast_to(r_row, (SPAD, N)))
         & (iota_s < SAMPLES)).astype(jnp.float32)

    q_red = _dot(p, Q, _HI)
    qk = _dot_nt(q_red, K) * (1.0 / math.sqrt(D))
    qk_max = jnp.max(qk, axis=-1, keepdims=True)
    e = jnp.exp(qk - qk_max)
    attn = e / jnp.sum(e, axis=-1, keepdims=True)

    attn_m = jnp.where(iota_s < SAMPLES, attn, -1.0)
    mx = jnp.max(attn_m, axis=0, keepdims=True)
    cp_row = jnp.min(jnp.where(attn_m == mx, iota_s, float(N)),
                     axis=0, keepdims=True)

    val = _dot(attn, V)

    cp_col = _dot_nt(ident, cp_row, _HI)
    iota_g = lax.broadcasted_iota(jnp.int32, (N, SPAD), 1).astype(jnp.float32)
    g = (jnp.broadcast_to(cp_col, (N, SPAD)) == iota_g).astype(jnp.float32)
    value = _dot(g, val, _HI)

    o1 = _dot(value, wo_ref[...]) + bo_ref[...] + x_
    o1 = _ln(o1)
    h = jnp.maximum(_dot(o1, wf1_ref[...]) + bf1_ref[...], 0.0)
    h = _dot(h, wf2_ref[...]) + bf2_ref[...]
    out_ref[u] = _ln(h + o1)


def kernel(x, adj, eigvec, eigvalue, Wq, bq, Wk, bk, Wv, bv, Wo, bo,
           Wp, bp, Wf1, bf1, Wf2, bf2):
    BT = B * T
    xr = x.reshape(BT, N, D)
    evT = jnp.transpose(eigvec[:, :, 0])
    ev = eigvalue.reshape(1, D)

    full = lambda shape: pl.BlockSpec(shape, lambda i: (0,) * len(shape))
    w = lambda: full((D, D))
    b = lambda: full((1, D))

    C = pl.pallas_call(
        _c_kernel,
        out_shape=jax.ShapeDtypeStruct((N, N), jnp.float32),
    )(adj)
    ident = jnp.eye(N, dtype=jnp.float32)

    out = pl.pallas_call(
        _fused_kernel,
        grid=(BT // UNROLL,),
        in_specs=[
            pl.BlockSpec((UNROLL, N, D), lambda i: (i, 0, 0)),
            full((N, N)),
            full((N, N)),
            full((N, D)),
            full((1, D)),
            w(), b(),
            w(), b(),
            w(), b(),
            w(), b(),
            full((1, D)), full((1, 1)),
            w(), b(),
            w(), b(),
        ],
        out_specs=pl.BlockSpec((UNROLL, N, D), lambda i: (i, 0, 0)),
        out_shape=jax.ShapeDtypeStruct((BT, N, D), jnp.float32),
        compiler_params=pltpu.CompilerParams(
            dimension_semantics=("parallel",)),
    )(xr, C, ident, evT, ev,
      Wq.T, bq.reshape(1, D), Wk.T, bk.reshape(1, D), Wv.T, bv.reshape(1, D),
      Wo.T, bo.reshape(1, D), Wp, bp.reshape(1, 1),
      Wf1.T, bf1.reshape(1, D), Wf2.T, bf2.reshape(1, D))
    return out.reshape(B, T, N, D)


# speedup vs baseline: 1.1111x; 1.0157x over previous
import math

import jax
import jax.numpy as jnp
from jax import lax
from jax.experimental import pallas as pl
from jax.experimental.pallas import tpu as pltpu

B, T, N, D = 8, 12, 512, 64
NADJ = 16
S_FACTOR = 5
SAMPLES = int(S_FACTOR * math.log(N, 2))
SPAD = 48

_HI = jax.lax.Precision.HIGHEST
_DEF = jax.lax.Precision.DEFAULT


def _dot(a, b, prec=_DEF):
    return lax.dot_general(a, b, (((1,), (0,)), ((), ())),
                           precision=prec, preferred_element_type=jnp.float32)


def _dot_nt(a, b, prec=_DEF):
    return lax.dot_general(a, b, (((1,), (1,)), ((), ())),
                           precision=prec, preferred_element_type=jnp.float32)


def _ln(x, eps=1e-5):
    mu = jnp.mean(x, axis=-1, keepdims=True)
    var = jnp.mean((x - mu) ** 2, axis=-1, keepdims=True)
    return (x - mu) / jnp.sqrt(var + eps)


def _c_kernel(adj_ref, c_ref):
    jI = lax.broadcasted_iota(jnp.int32, (N, N), 1)
    acc = jnp.zeros((N, N), jnp.float32)
    for k in range(NADJ):
        a = adj_ref[:, k][:, None]
        acc = acc + (a == jI).astype(jnp.float32)
    c_ref[...] = acc


UNROLL = 4


def _fused_kernel(x_ref, c_ref, ident_ref, evT_ref, ev_ref,
                  wq_ref, bq_ref, wk_ref, bk_ref, wv_ref, bv_ref,
                  wo_ref, bo_ref, wp_ref, bp_ref,
                  wf1_ref, bf1_ref, wf2_ref, bf2_ref,
                  out_ref):
    ident = ident_ref[...]
    for u in range(UNROLL):
        _one_slice(u, x_ref, c_ref, ident, evT_ref, ev_ref,
                   wq_ref, bq_ref, wk_ref, bk_ref, wv_ref, bv_ref,
                   wo_ref, bo_ref, wp_ref, bp_ref,
                   wf1_ref, bf1_ref, wf2_ref, bf2_ref, out_ref)


def _one_slice(u, x_ref, c_ref, ident, evT_ref, ev_ref,
               wq_ref, bq_ref, wk_ref, bk_ref, wv_ref, bv_ref,
               wo_ref, bo_ref, wp_ref, bp_ref,
               wf1_ref, bf1_ref, wf2_ref, bf2_ref, out_ref):
    x = x_ref[u]
    identd = (lax.broadcasted_iota(jnp.int32, (D, D), 0)
              == lax.broadcasted_iota(jnp.int32, (D, D), 1)).astype(jnp.float32)
    diagm = identd * ev_ref[...]
    pe = _dot(evT_ref[...], diagm)
    x_ = x + pe

    Q = _dot(x_, wq_ref[...]) + bq_ref[...]
    K = _dot(x_, wk_ref[...]) + bk_ref[...]
    V = _dot(x_, wv_ref[...]) + bv_ref[...]

    S_hi = _dot_nt(Q, K, _HI)
    gat = _dot(S_hi * c_ref[...], V, _HI)

    m_row = _dot_nt(wp_ref[...], gat) + bp_ref[...]
    m_col = _dot_nt(ident, m_row, _HI)

    mj = jnp.broadcast_to(m_col, (N, N))
    mn = jnp.broadcast_to(m_row, (N, N))
    ij = lax.broadcasted_iota(jnp.int32, (N, N), 0)
    inn = lax.broadcasted_iota(jnp.int32, (N, N), 1)
    gt = (mj > mn) | ((mj == mn) & (ij < inn))
    r_row = _dot(jnp.ones((1, N), jnp.float32),
                 gt.astype(jnp.float32))

    iota_s = lax.broadcasted_iota(jnp.int32, (SPAD, N), 0).astype(jnp.float32)
    p = ((iota_s == jnp.broadcast_to(r_row, (SPAD, N)))
         & (iota_s < SAMPLES)).astype(jnp.float32)

    q_red = _dot(p, Q, _HI)
    qk = _dot_nt(q_red, K) * (1.0 / math.sqrt(D))
    qk_max = jnp.max(qk, axis=-1, keepdims=True)
    e = jnp.exp(qk - qk_max)
    attn = e / jnp.sum(e, axis=-1, keepdims=True)

    attn_m = jnp.where(iota_s < SAMPLES, attn, -1.0)
    mx = jnp.max(attn_m, axis=0, keepdims=True)
    cp_row = jnp.min(jnp.where(attn_m == mx, iota_s, float(N)),
                     axis=0, keepdims=True)

    val = _dot(attn, V)

    cp_col = _dot_nt(ident, cp_row, _HI)
    iota_g = lax.broadcasted_iota(jnp.int32, (N, SPAD), 1).astype(jnp.float32)
    g = (jnp.broadcast_to(cp_col, (N, SPAD)) == iota_g).astype(jnp.float32)
    value = _dot(g, val, _HI)

    o1 = _dot(value, wo_ref[...]) + bo_ref[...] + x_
    o1 = _ln(o1)
    h = jnp.maximum(_dot(o1, wf1_ref[...]) + bf1_ref[...], 0.0)
    h = _dot(h, wf2_ref[...]) + bf2_ref[...]
    out_ref[u] = _ln(h + o1)


def kernel(x, adj, eigvec, eigvalue, Wq, bq, Wk, bk, Wv, bv, Wo, bo,
           Wp, bp, Wf1, bf1, Wf2, bf2):
    BT = B * T
    xr = x.reshape(BT, N, D)
    evT = jnp.transpose(eigvec[:, :, 0])
    ev = eigvalue.reshape(1, D)

    full = lambda shape: pl.BlockSpec(shape, lambda i: (0,) * len(shape))
    w = lambda: full((D, D))
    b = lambda: full((1, D))

    C = pl.pallas_call(
        _c_kernel,
        out_shape=jax.ShapeDtypeStruct((N, N), jnp.float32),
    )(adj)
    ident = jnp.eye(N, dtype=jnp.float32)

    out = pl.pallas_call(
        _fused_kernel,
        grid=(BT // UNROLL,),
        in_specs=[
            pl.BlockSpec((UNROLL, N, D), lambda i: (i, 0, 0)),
            full((N, N)),
            full((N, N)),
            full((N, D)),
            full((1, D)),
            w(), b(),
            w(), b(),
            w(), b(),
            w(), b(),
            full((1, D)), full((1, 1)),
            w(), b(),
            w(), b(),
        ],
        out_specs=pl.BlockSpec((UNROLL, N, D), lambda i: (i, 0, 0)),
        out_shape=jax.ShapeDtypeStruct((BT, N, D), jnp.float32),
        compiler_params=pltpu.CompilerParams(
            dimension_semantics=("parallel",)),
    )(xr, C, ident, evT, ev,
      Wq.T, bq.reshape(1, D), Wk.T, bk.reshape(1, D), Wv.T, bv.reshape(1, D),
      Wo.T, bo.reshape(1, D), Wp, bp.reshape(1, 1),
      Wf1.T, bf1.reshape(1, D), Wf2.T, bf2.reshape(1, D))
    return out.reshape(B, T, N, D)


# speedup vs baseline: 1.1204x; 1.0084x over previous
import math

import jax
import jax.numpy as jnp
from jax import lax
from jax.experimental import pallas as pl
from jax.experimental.pallas import tpu as pltpu

B, T, N, D = 8, 12, 512, 64
NADJ = 16
S_FACTOR = 5
SAMPLES = int(S_FACTOR * math.log(N, 2))
SPAD = 48

_HI = jax.lax.Precision.HIGHEST
_DEF = jax.lax.Precision.DEFAULT


def _dot(a, b, prec=_DEF):
    return lax.dot_general(a, b, (((1,), (0,)), ((), ())),
                           precision=prec, preferred_element_type=jnp.float32)


def _dot_nt(a, b, prec=_DEF):
    return lax.dot_general(a, b, (((1,), (1,)), ((), ())),
                           precision=prec, preferred_element_type=jnp.float32)


def _ln(x, eps=1e-5):
    mu = jnp.mean(x, axis=-1, keepdims=True)
    var = jnp.mean((x - mu) ** 2, axis=-1, keepdims=True)
    return (x - mu) / jnp.sqrt(var + eps)


def _c_kernel(adj_ref, c_ref):
    jI = lax.broadcasted_iota(jnp.int32, (N, N), 1)
    acc = jnp.zeros((N, N), jnp.float32)
    for k in range(NADJ):
        a = adj_ref[:, k][:, None]
        acc = acc + (a == jI).astype(jnp.float32)
    c_ref[...] = acc


UNROLL = 6


def _fused_kernel(x_ref, c_ref, ident_ref, evT_ref, ev_ref,
                  wq_ref, bq_ref, wk_ref, bk_ref, wv_ref, bv_ref,
                  wo_ref, bo_ref, wp_ref, bp_ref,
                  wf1_ref, bf1_ref, wf2_ref, bf2_ref,
                  out_ref):
    ident = ident_ref[...]
    for u in range(UNROLL):
        _one_slice(u, x_ref, c_ref, ident, evT_ref, ev_ref,
                   wq_ref, bq_ref, wk_ref, bk_ref, wv_ref, bv_ref,
                   wo_ref, bo_ref, wp_ref, bp_ref,
                   wf1_ref, bf1_ref, wf2_ref, bf2_ref, out_ref)


def _one_slice(u, x_ref, c_ref, ident, evT_ref, ev_ref,
               wq_ref, bq_ref, wk_ref, bk_ref, wv_ref, bv_ref,
               wo_ref, bo_ref, wp_ref, bp_ref,
               wf1_ref, bf1_ref, wf2_ref, bf2_ref, out_ref):
    x = x_ref[u]
    identd = (lax.broadcasted_iota(jnp.int32, (D, D), 0)
              == lax.broadcasted_iota(jnp.int32, (D, D), 1)).astype(jnp.float32)
    diagm = identd * ev_ref[...]
    pe = _dot(evT_ref[...], diagm)
    x_ = x + pe

    Q = _dot(x_, wq_ref[...]) + bq_ref[...]
    K = _dot(x_, wk_ref[...]) + bk_ref[...]
    V = _dot(x_, wv_ref[...]) + bv_ref[...]

    S_hi = _dot_nt(Q, K, _HI)
    gat = _dot(S_hi * c_ref[...], V, _HI)

    m_row = _dot_nt(wp_ref[...], gat) + bp_ref[...]
    m_col = _dot_nt(ident, m_row, _HI)

    mj = jnp.broadcast_to(m_col, (N, N))
    mn = jnp.broadcast_to(m_row, (N, N))
    ij = lax.broadcasted_iota(jnp.int32, (N, N), 0)
    inn = lax.broadcasted_iota(jnp.int32, (N, N), 1)
    gt = (mj > mn) | ((mj == mn) & (ij < inn))
    r_row = _dot(jnp.ones((1, N), jnp.float32),
                 gt.astype(jnp.float32))

    iota_s = lax.broadcasted_iota(jnp.int32, (SPAD, N), 0).astype(jnp.float32)
    p = ((iota_s == jnp.broadcast_to(r_row, (SPAD, N)))
         & (iota_s < SAMPLES)).astype(jnp.float32)

    q_red = _dot(p, Q, _HI)
    qk = _dot_nt(q_red, K) * (1.0 / math.sqrt(D))
    qk_max = jnp.max(qk, axis=-1, keepdims=True)
    e = jnp.exp(qk - qk_max)
    attn = e / jnp.sum(e, axis=-1, keepdims=True)

    attn_m = jnp.where(iota_s < SAMPLES, attn, -1.0)
    mx = jnp.max(attn_m, axis=0, keepdims=True)
    cp_row = jnp.min(jnp.where(attn_m == mx, iota_s, float(N)),
                     axis=0, keepdims=True)

    val = _dot(attn, V)

    cp_col = _dot_nt(ident, cp_row, _HI)
    iota_g = lax.broadcasted_iota(jnp.int32, (N, SPAD), 1).astype(jnp.float32)
    g = (jnp.broadcast_to(cp_col, (N, SPAD)) == iota_g).astype(jnp.float32)
    value = _dot(g, val, _HI)

    o1 = _dot(value, wo_ref[...]) + bo_ref[...] + x_
    o1 = _ln(o1)
    h = jnp.maximum(_dot(o1, wf1_ref[...]) + bf1_ref[...], 0.0)
    h = _dot(h, wf2_ref[...]) + bf2_ref[...]
    out_ref[u] = _ln(h + o1)


def kernel(x, adj, eigvec, eigvalue, Wq, bq, Wk, bk, Wv, bv, Wo, bo,
           Wp, bp, Wf1, bf1, Wf2, bf2):
    BT = B * T
    xr = x.reshape(BT, N, D)
    evT = jnp.transpose(eigvec[:, :, 0])
    ev = eigvalue.reshape(1, D)

    full = lambda shape: pl.BlockSpec(shape, lambda i: (0,) * len(shape))
    w = lambda: full((D, D))
    b = lambda: full((1, D))

    C = pl.pallas_call(
        _c_kernel,
        out_shape=jax.ShapeDtypeStruct((N, N), jnp.float32),
    )(adj)
    ident = jnp.eye(N, dtype=jnp.float32)

    out = pl.pallas_call(
        _fused_kernel,
        grid=(BT // UNROLL,),
        in_specs=[
            pl.BlockSpec((UNROLL, N, D), lambda i: (i, 0, 0)),
            full((N, N)),
            full((N, N)),
            full((N, D)),
            full((1, D)),
            w(), b(),
            w(), b(),
            w(), b(),
            w(), b(),
            full((1, D)), full((1, 1)),
            w(), b(),
            w(), b(),
        ],
        out_specs=pl.BlockSpec((UNROLL, N, D), lambda i: (i, 0, 0)),
        out_shape=jax.ShapeDtypeStruct((BT, N, D), jnp.float32),
        compiler_params=pltpu.CompilerParams(
            dimension_semantics=("parallel",)),
    )(xr, C, ident, evT, ev,
      Wq.T, bq.reshape(1, D), Wk.T, bk.reshape(1, D), Wv.T, bv.reshape(1, D),
      Wo.T, bo.reshape(1, D), Wp, bp.reshape(1, 1),
      Wf1.T, bf1.reshape(1, D), Wf2.T, bf2.reshape(1, D))
    return out.reshape(B, T, N, D)


# speedup vs baseline: 1.2074x; 1.0776x over previous
import math

import jax
import jax.numpy as jnp
from jax import lax
from jax.experimental import pallas as pl
from jax.experimental.pallas import tpu as pltpu

B, T, N, D = 8, 12, 512, 64
NADJ = 16
S_FACTOR = 5
SAMPLES = int(S_FACTOR * math.log(N, 2))
SPAD = 48

_HI = jax.lax.Precision.HIGHEST
_DEF = jax.lax.Precision.DEFAULT


def _dot(a, b, prec=_DEF):
    return lax.dot_general(a, b, (((1,), (0,)), ((), ())),
                           precision=prec, preferred_element_type=jnp.float32)


def _dot_nt(a, b, prec=_DEF):
    return lax.dot_general(a, b, (((1,), (1,)), ((), ())),
                           precision=prec, preferred_element_type=jnp.float32)


def _ln(x, eps=1e-5):
    mu = jnp.mean(x, axis=-1, keepdims=True)
    var = jnp.mean((x - mu) ** 2, axis=-1, keepdims=True)
    return (x - mu) / jnp.sqrt(var + eps)


def _c_kernel(adj_ref, c_ref):
    jI = lax.broadcasted_iota(jnp.int32, (N, N), 1)
    acc = jnp.zeros((N, N), jnp.float32)
    for k in range(NADJ):
        a = adj_ref[:, k][:, None]
        acc = acc + (a == jI).astype(jnp.float32)
    c_ref[...] = acc


UNROLL = 6


def _fused_kernel(x_ref, c_ref, ident_ref, evT_ref, ev_ref,
                  wq_ref, bq_ref, wk_ref, bk_ref, wv_ref, bv_ref,
                  wo_ref, bo_ref, wp_ref, bp_ref,
                  wf1_ref, bf1_ref, wf2_ref, bf2_ref,
                  out_ref):
    ident = ident_ref[...]
    for u in range(UNROLL):
        _one_slice(u, x_ref, c_ref, ident, evT_ref, ev_ref,
                   wq_ref, bq_ref, wk_ref, bk_ref, wv_ref, bv_ref,
                   wo_ref, bo_ref, wp_ref, bp_ref,
                   wf1_ref, bf1_ref, wf2_ref, bf2_ref, out_ref)


def _one_slice(u, x_ref, c_ref, ident, evT_ref, ev_ref,
               wq_ref, bq_ref, wk_ref, bk_ref, wv_ref, bv_ref,
               wo_ref, bo_ref, wp_ref, bp_ref,
               wf1_ref, bf1_ref, wf2_ref, bf2_ref, out_ref):
    x = x_ref[u]
    identd = (lax.broadcasted_iota(jnp.int32, (D, D), 0)
              == lax.broadcasted_iota(jnp.int32, (D, D), 1)).astype(jnp.float32)
    diagm = identd * ev_ref[...]
    pe = _dot(evT_ref[...], diagm)
    x_ = x + pe

    Q = _dot(x_, wq_ref[...]) + bq_ref[...]
    K = _dot(x_, wk_ref[...]) + bk_ref[...]
    V = _dot(x_, wv_ref[...]) + bv_ref[...]

    S_hi = _dot_nt(Q, K, _HI)
    gat = _dot(S_hi * c_ref[...], V, _HI)

    m_row = _dot_nt(wp_ref[...], gat) + bp_ref[...]
    m_col = _dot_nt(ident, m_row, _HI)

    mj = jnp.broadcast_to(m_col, (N, N))
    mn = jnp.broadcast_to(m_row, (N, N))
    ij = lax.broadcasted_iota(jnp.int32, (N, N), 0)
    inn = lax.broadcasted_iota(jnp.int32, (N, N), 1)
    gt = (mj > mn) | ((mj == mn) & (ij < inn))
    r_row = _dot(jnp.ones((1, N), jnp.float32),
                 gt.astype(jnp.float32))

    iota_s = lax.broadcasted_iota(jnp.int32, (SPAD, N), 0).astype(jnp.float32)
    p = ((iota_s == jnp.broadcast_to(r_row, (SPAD, N)))
         & (iota_s < SAMPLES)).astype(jnp.float32)

    q_red = _dot(p, Q)
    qk = _dot_nt(q_red, K) * (1.0 / math.sqrt(D))
    qk_max = jnp.max(qk, axis=-1, keepdims=True)
    e = jnp.exp(qk - qk_max)
    attn = e / jnp.sum(e, axis=-1, keepdims=True)

    attn_m = jnp.where(iota_s < SAMPLES, attn, -1.0)
    mx = jnp.max(attn_m, axis=0, keepdims=True)
    cp_row = jnp.min(jnp.where(attn_m == mx, iota_s, float(N)),
                     axis=0, keepdims=True)

    val = _dot(attn, V)

    cp_col = _dot_nt(ident, cp_row)
    iota_g = lax.broadcasted_iota(jnp.int32, (N, SPAD), 1).astype(jnp.float32)
    g = (jnp.broadcast_to(cp_col, (N, SPAD)) == iota_g).astype(jnp.float32)
    value = _dot(g, val)

    o1 = _dot(value, wo_ref[...]) + bo_ref[...] + x_
    o1 = _ln(o1)
    h = jnp.maximum(_dot(o1, wf1_ref[...]) + bf1_ref[...], 0.0)
    h = _dot(h, wf2_ref[...]) + bf2_ref[...]
    out_ref[u] = _ln(h + o1)


def kernel(x, adj, eigvec, eigvalue, Wq, bq, Wk, bk, Wv, bv, Wo, bo,
           Wp, bp, Wf1, bf1, Wf2, bf2):
    BT = B * T
    xr = x.reshape(BT, N, D)
    evT = jnp.transpose(eigvec[:, :, 0])
    ev = eigvalue.reshape(1, D)

    full = lambda shape: pl.BlockSpec(shape, lambda i: (0,) * len(shape))
    w = lambda: full((D, D))
    b = lambda: full((1, D))

    C = pl.pallas_call(
        _c_kernel,
        out_shape=jax.ShapeDtypeStruct((N, N), jnp.float32),
    )(adj)
    ident = jnp.eye(N, dtype=jnp.float32)

    out = pl.pallas_call(
        _fused_kernel,
        grid=(BT // UNROLL,),
        in_specs=[
            pl.BlockSpec((UNROLL, N, D), lambda i: (i, 0, 0)),
            full((N, N)),
            full((N, N)),
            full((N, D)),
            full((1, D)),
            w(), b(),
            w(), b(),
            w(), b(),
            w(), b(),
            full((1, D)), full((1, 1)),
            w(), b(),
            w(), b(),
        ],
        out_specs=pl.BlockSpec((UNROLL, N, D), lambda i: (i, 0, 0)),
        out_shape=jax.ShapeDtypeStruct((BT, N, D), jnp.float32),
        compiler_params=pltpu.CompilerParams(
            dimension_semantics=("parallel",)),
    )(xr, C, ident, evT, ev,
      Wq.T, bq.reshape(1, D), Wk.T, bk.reshape(1, D), Wv.T, bv.reshape(1, D),
      Wo.T, bo.reshape(1, D), Wp, bp.reshape(1, 1),
      Wf1.T, bf1.reshape(1, D), Wf2.T, bf2.reshape(1, D))
    return out.reshape(B, T, N, D)


# speedup vs baseline: 1.2128x; 1.0044x over previous
import math

import jax
import jax.numpy as jnp
from jax import lax
from jax.experimental import pallas as pl
from jax.experimental.pallas import tpu as pltpu

B, T, N, D = 8, 12, 512, 64
NADJ = 16
S_FACTOR = 5
SAMPLES = int(S_FACTOR * math.log(N, 2))
SPAD = 48

_HI = jax.lax.Precision.HIGHEST
_DEF = jax.lax.Precision.DEFAULT


def _dot(a, b, prec=_DEF):
    return lax.dot_general(a, b, (((1,), (0,)), ((), ())),
                           precision=prec, preferred_element_type=jnp.float32)


def _dot_nt(a, b, prec=_DEF):
    return lax.dot_general(a, b, (((1,), (1,)), ((), ())),
                           precision=prec, preferred_element_type=jnp.float32)


def _ln(x, eps=1e-5):
    mu = jnp.mean(x, axis=-1, keepdims=True)
    var = jnp.mean((x - mu) ** 2, axis=-1, keepdims=True)
    return (x - mu) / jnp.sqrt(var + eps)


def _c_kernel(adj_ref, c_ref):
    jI = lax.broadcasted_iota(jnp.int32, (N, N), 1)
    acc = jnp.zeros((N, N), jnp.float32)
    for k in range(NADJ):
        a = adj_ref[:, k][:, None]
        acc = acc + (a == jI).astype(jnp.float32)
    c_ref[...] = acc


UNROLL = 8


def _fused_kernel(x_ref, c_ref, ident_ref, evT_ref, ev_ref,
                  wq_ref, bq_ref, wk_ref, bk_ref, wv_ref, bv_ref,
                  wo_ref, bo_ref, wp_ref, bp_ref,
                  wf1_ref, bf1_ref, wf2_ref, bf2_ref,
                  out_ref):
    ident = ident_ref[...]
    for u in range(UNROLL):
        _one_slice(u, x_ref, c_ref, ident, evT_ref, ev_ref,
                   wq_ref, bq_ref, wk_ref, bk_ref, wv_ref, bv_ref,
                   wo_ref, bo_ref, wp_ref, bp_ref,
                   wf1_ref, bf1_ref, wf2_ref, bf2_ref, out_ref)


def _one_slice(u, x_ref, c_ref, ident, evT_ref, ev_ref,
               wq_ref, bq_ref, wk_ref, bk_ref, wv_ref, bv_ref,
               wo_ref, bo_ref, wp_ref, bp_ref,
               wf1_ref, bf1_ref, wf2_ref, bf2_ref, out_ref):
    x = x_ref[u]
    identd = (lax.broadcasted_iota(jnp.int32, (D, D), 0)
              == lax.broadcasted_iota(jnp.int32, (D, D), 1)).astype(jnp.float32)
    diagm = identd * ev_ref[...]
    pe = _dot(evT_ref[...], diagm)
    x_ = x + pe

    Q = _dot(x_, wq_ref[...]) + bq_ref[...]
    K = _dot(x_, wk_ref[...]) + bk_ref[...]
    V = _dot(x_, wv_ref[...]) + bv_ref[...]

    S_hi = _dot_nt(Q, K, _HI)
    gat = _dot(S_hi * c_ref[...], V, _HI)

    m_row = _dot_nt(wp_ref[...], gat) + bp_ref[...]
    m_col = _dot_nt(ident, m_row, _HI)

    mj = jnp.broadcast_to(m_col, (N, N))
    mn = jnp.broadcast_to(m_row, (N, N))
    ij = lax.broadcasted_iota(jnp.int32, (N, N), 0)
    inn = lax.broadcasted_iota(jnp.int32, (N, N), 1)
    gt = (mj > mn) | ((mj == mn) & (ij < inn))
    r_row = _dot(jnp.ones((1, N), jnp.float32),
                 gt.astype(jnp.float32))

    iota_s = lax.broadcasted_iota(jnp.int32, (SPAD, N), 0).astype(jnp.float32)
    p = ((iota_s == jnp.broadcast_to(r_row, (SPAD, N)))
         & (iota_s < SAMPLES)).astype(jnp.float32)

    q_red = _dot(p, Q)
    qk = _dot_nt(q_red, K) * (1.0 / math.sqrt(D))
    qk_max = jnp.max(qk, axis=-1, keepdims=True)
    e = jnp.exp(qk - qk_max)
    attn = e / jnp.sum(e, axis=-1, keepdims=True)

    attn_m = jnp.where(iota_s < SAMPLES, attn, -1.0)
    mx = jnp.max(attn_m, axis=0, keepdims=True)
    cp_row = jnp.min(jnp.where(attn_m == mx, iota_s, float(N)),
                     axis=0, keepdims=True)

    val = _dot(attn, V)

    cp_col = _dot_nt(ident, cp_row)
    iota_g = lax.broadcasted_iota(jnp.int32, (N, SPAD), 1).astype(jnp.float32)
    g = (jnp.broadcast_to(cp_col, (N, SPAD)) == iota_g).astype(jnp.float32)
    value = _dot(g, val)

    o1 = _dot(value, wo_ref[...]) + bo_ref[...] + x_
    o1 = _ln(o1)
    h = jnp.maximum(_dot(o1, wf1_ref[...]) + bf1_ref[...], 0.0)
    h = _dot(h, wf2_ref[...]) + bf2_ref[...]
    out_ref[u] = _ln(h + o1)


def kernel(x, adj, eigvec, eigvalue, Wq, bq, Wk, bk, Wv, bv, Wo, bo,
           Wp, bp, Wf1, bf1, Wf2, bf2):
    BT = B * T
    xr = x.reshape(BT, N, D)
    evT = jnp.transpose(eigvec[:, :, 0])
    ev = eigvalue.reshape(1, D)

    full = lambda shape: pl.BlockSpec(shape, lambda i: (0,) * len(shape))
    w = lambda: full((D, D))
    b = lambda: full((1, D))

    C = pl.pallas_call(
        _c_kernel,
        out_shape=jax.ShapeDtypeStruct((N, N), jnp.float32),
    )(adj)
    ident = jnp.eye(N, dtype=jnp.float32)

    out = pl.pallas_call(
        _fused_kernel,
        grid=(BT // UNROLL,),
        in_specs=[
            pl.BlockSpec((UNROLL, N, D), lambda i: (i, 0, 0)),
            full((N, N)),
            full((N, N)),
            full((N, D)),
            full((1, D)),
            w(), b(),
            w(), b(),
            w(), b(),
            w(), b(),
            full((1, D)), full((1, 1)),
            w(), b(),
            w(), b(),
        ],
        out_specs=pl.BlockSpec((UNROLL, N, D), lambda i: (i, 0, 0)),
        out_shape=jax.ShapeDtypeStruct((BT, N, D), jnp.float32),
        compiler_params=pltpu.CompilerParams(
            dimension_semantics=("parallel",)),
    )(xr, C, ident, evT, ev,
      Wq.T, bq.reshape(1, D), Wk.T, bk.reshape(1, D), Wv.T, bv.reshape(1, D),
      Wo.T, bo.reshape(1, D), Wp, bp.reshape(1, 1),
      Wf1.T, bf1.reshape(1, D), Wf2.T, bf2.reshape(1, D))
    return out.reshape(B, T, N, D)
